# baseline (speedup 1.0000x reference)
import functools

import jax
import jax.numpy as jnp
from jax import lax
from jax.experimental import pallas as pl
from jax.experimental.pallas import tpu as pltpu
from jax.experimental.pallas import tpu_sc as plsc

_E = 8
_I = 4096
_H = 2048
_T = 2048
_K = 2
_A = _T * _K

_B = 256
_C = _A + _E * _B
_NB = _C // _B
_IB = 512
_JB = _I // _IB

_NW = 32


def _routing_metadata(top_k_index, top_k_weights, per_expert_scale):
    e_flat = top_k_index.reshape(-1).astype(jnp.int32)
    oh = (e_flat[:, None] == jnp.arange(_E, dtype=jnp.int32)[None, :])
    oh_i = oh.astype(jnp.int32)
    w_flat = (top_k_weights.reshape(-1)
              * (oh.astype(jnp.float32) @ per_expert_scale))
    ranks = jnp.cumsum(oh_i, axis=0) - oh_i
    counts = jnp.sum(oh_i, axis=0)
    blocks_per_e = (counts + _B - 1) // _B
    block_off = jnp.cumsum(blocks_per_e) - blocks_per_e
    total_blocks = block_off[-1] + blocks_per_e[-1]
    pos = (block_off[e_flat] * _B
           + jnp.sum(ranks * oh_i, axis=1)).astype(jnp.int32)
    tok = (jnp.arange(_A, dtype=jnp.int32) // _K)
    tok_sorted = jnp.zeros((_C,), jnp.int32).at[pos].set(tok)
    w_sorted = jnp.zeros((_C,), jnp.float32).at[pos].set(w_flat)
    bgrid = jnp.arange(_NB, dtype=jnp.int32)
    block_expert = (jnp.sum(bgrid[:, None] >= block_off[None, :], axis=1)
                    .astype(jnp.int32) - 1)
    block_valid = (bgrid < total_blocks).astype(jnp.int32)
    pos2 = pos.reshape(_T, _K)
    return tok_sorted, w_sorted, block_expert, block_valid, pos2[:, 0], pos2[:, 1]


_ROWS_PER_W = _C // _NW
_GCHUNK = 48


def _sc_gather_body(tok_hbm, hid_hbm, x_hbm, idx_v, rows_v, sem):
    wid = lax.axis_index("s") * 2 + lax.axis_index("c")
    base = wid * _ROWS_PER_W
    for c in range(_ROWS_PER_W // _GCHUNK):
        off = base + c * _GCHUNK
        pltpu.sync_copy(tok_hbm.at[pl.ds(off, _GCHUNK)], idx_v)
        pltpu.async_copy(hid_hbm.at[idx_v], rows_v, sem).wait()
        pltpu.sync_copy(rows_v, x_hbm.at[pl.ds(off, _GCHUNK)])


def _sc_gather(tok_sorted, hidden_states):
    mesh = plsc.VectorSubcoreMesh(core_axis_name="c", subcore_axis_name="s")
    fn = functools.partial(
        pl.kernel,
        mesh=mesh,
        out_type=jax.ShapeDtypeStruct((_C, _H), jnp.float32),
        scratch_types=[
            pltpu.VMEM((_GCHUNK,), jnp.int32),
            pltpu.VMEM((_GCHUNK, _H), jnp.float32),
            pltpu.SemaphoreType.DMA,
        ],
    )(_sc_gather_body)
    return fn(tok_sorted, hidden_states)


def _mlp_body(be_ref, bv_ref, x_ref, g_ref, u_ref, d_ref, w_ref, out_ref):
    s = pl.program_id(0)
    j = pl.program_id(1)

    @pl.when(bv_ref[s] == 1)
    def _():
        @pl.when(j == 0)
        def _():
            out_ref[...] = jnp.zeros_like(out_ref)

        x = x_ref[...]
        g = lax.dot_general(x, g_ref[0], (((1,), (1,)), ((), ())),
                            preferred_element_type=jnp.float32)
        u = lax.dot_general(x, u_ref[0], (((1,), (1,)), ((), ())),
                            preferred_element_type=jnp.float32)
        h = g * lax.logistic(g) * u
        h = h * w_ref[...]
        out_ref[...] += lax.dot_general(h, d_ref[0], (((1,), (1,)), ((), ())),
                                        preferred_element_type=jnp.float32)


def _tc_grouped_mlp(block_expert, block_valid, x_sorted, gate_up_proj,
                    down_proj, w_sorted):
    w2 = w_sorted.reshape(_C, 1)

    def gmap(s, j, be, bv):
        return (be[s], jnp.where(bv[s] == 1, j, _JB - 1), 0)

    def umap(s, j, be, bv):
        return (be[s], _JB + jnp.where(bv[s] == 1, j, _JB - 1), 0)

    def dmap(s, j, be, bv):
        return (be[s], 0, jnp.where(bv[s] == 1, j, _JB - 1))

    grid_spec = pltpu.PrefetchScalarGridSpec(
        num_scalar_prefetch=2,
        grid=(_NB, _JB),
        in_specs=[
            pl.BlockSpec((_B, _H), lambda s, j, be, bv: (s, 0)),
            pl.BlockSpec((1, _IB, _H), gmap),
            pl.BlockSpec((1, _IB, _H), umap),
            pl.BlockSpec((1, _H, _IB), dmap),
            pl.BlockSpec((_B, 1), lambda s, j, be, bv: (s, 0)),
        ],
        out_specs=pl.BlockSpec((_B, _H), lambda s, j, be, bv: (s, 0)),
    )
    return pl.pallas_call(
        _mlp_body,
        grid_spec=grid_spec,
        out_shape=jax.ShapeDtypeStruct((_C, _H), jnp.float32),
    )(block_expert, block_valid, x_sorted, gate_up_proj, gate_up_proj,
      down_proj, w2)


_TOK_PER_W = _T // _NW
_CCHUNK = 16


def _sc_combine_body(p0_hbm, p1_hbm, osort_hbm, fin_hbm,
                     i0_v, i1_v, r0_v, r1_v, sem0, sem1):
    wid = lax.axis_index("s") * 2 + lax.axis_index("c")
    base = wid * _TOK_PER_W
    for c in range(_TOK_PER_W // _CCHUNK):
        off = base + c * _CCHUNK
        pltpu.sync_copy(p0_hbm.at[pl.ds(off, _CCHUNK)], i0_v)
        pltpu.sync_copy(p1_hbm.at[pl.ds(off, _CCHUNK)], i1_v)
        cp0 = pltpu.async_copy(osort_hbm.at[i0_v], r0_v, sem0)
        cp1 = pltpu.async_copy(osort_hbm.at[i1_v], r1_v, sem1)
        cp0.wait()
        cp1.wait()

        def row_body(r, carry):
            for cc in range(_H // 16):
                sl = pl.ds(cc * 16, 16)
                r0_v[r, sl] = r0_v[r, sl] + r1_v[r, sl]
            return carry

        lax.fori_loop(0, _CCHUNK, row_body, 0)
        pltpu.sync_copy(r0_v, fin_hbm.at[pl.ds(off, _CCHUNK)])


def _sc_combine(pos0, pos1, out_sorted):
    mesh = plsc.VectorSubcoreMesh(core_axis_name="c", subcore_axis_name="s")
    fn = functools.partial(
        pl.kernel,
        mesh=mesh,
        out_type=jax.ShapeDtypeStruct((_T, _H), jnp.float32),
        scratch_types=[
            pltpu.VMEM((_CCHUNK,), jnp.int32),
            pltpu.VMEM((_CCHUNK,), jnp.int32),
            pltpu.VMEM((_CCHUNK, _H), jnp.float32),
            pltpu.VMEM((_CCHUNK, _H), jnp.float32),
            pltpu.SemaphoreType.DMA,
            pltpu.SemaphoreType.DMA,
        ],
    )(_sc_combine_body)
    return fn(pos0, pos1, out_sorted)


def kernel(hidden_states, top_k_index, top_k_weights, gate_up_proj, down_proj,
           per_expert_scale):
    (tok_sorted, w_sorted, block_expert, block_valid,
     pos0, pos1) = _routing_metadata(top_k_index, top_k_weights,
                                     per_expert_scale)
    x_sorted = _sc_gather(tok_sorted, hidden_states)
    out_sorted = _tc_grouped_mlp(block_expert, block_valid, x_sorted,
                                 gate_up_proj, down_proj, w_sorted)
    return _sc_combine(pos0, pos1, out_sorted)


# speedup vs baseline: 1.0241x; 1.0241x over previous
import functools

import jax
import jax.numpy as jnp
from jax import lax
from jax.experimental import pallas as pl
from jax.experimental.pallas import tpu as pltpu
from jax.experimental.pallas import tpu_sc as plsc

_E = 8
_I = 4096
_H = 2048
_T = 2048
_K = 2
_A = _T * _K

_B = 512
_C = _A + _E * _B
_NB = _C // _B
_IB = 512
_JB = _I // _IB

_NW = 32


def _routing_metadata(top_k_index, top_k_weights, per_expert_scale):
    e_flat = top_k_index.reshape(-1).astype(jnp.int32)
    oh = (e_flat[:, None] == jnp.arange(_E, dtype=jnp.int32)[None, :])
    oh_i = oh.astype(jnp.int32)
    w_flat = (top_k_weights.reshape(-1)
              * (oh.astype(jnp.float32) @ per_expert_scale))
    ranks = jnp.cumsum(oh_i, axis=0) - oh_i
    counts = jnp.sum(oh_i, axis=0)
    blocks_per_e = (counts + _B - 1) // _B
    block_off = jnp.cumsum(blocks_per_e) - blocks_per_e
    total_blocks = block_off[-1] + blocks_per_e[-1]
    pos = (block_off[e_flat] * _B
           + jnp.sum(ranks * oh_i, axis=1)).astype(jnp.int32)
    tok = (jnp.arange(_A, dtype=jnp.int32) // _K)
    tok_sorted = jnp.zeros((_C,), jnp.int32).at[pos].set(tok)
    w_sorted = jnp.zeros((_C,), jnp.float32).at[pos].set(w_flat)
    bgrid = jnp.arange(_NB, dtype=jnp.int32)
    block_expert = (jnp.sum(bgrid[:, None] >= block_off[None, :], axis=1)
                    .astype(jnp.int32) - 1)
    block_valid = (bgrid < total_blocks).astype(jnp.int32)
    pos2 = pos.reshape(_T, _K)
    return tok_sorted, w_sorted, block_expert, block_valid, pos2[:, 0], pos2[:, 1]


_ROWS_PER_W = _C // _NW
_GCHUNK = 16
_GN = _ROWS_PER_W // _GCHUNK


def _sc_gather_body(tok_hbm, hid_hbm, x_hbm, idx_v, rows_v, gsem, wsem):
    wid = lax.axis_index("s") * 2 + lax.axis_index("c")
    base = wid * _ROWS_PER_W

    pltpu.sync_copy(tok_hbm.at[pl.ds(base, _GCHUNK)], idx_v.at[0])
    g_prev = pltpu.async_copy(hid_hbm.at[idx_v.at[0]], rows_v.at[0], gsem)
    w_handles = [None, None]
    for c in range(_GN):
        cur = c % 2
        nxt = (c + 1) % 2
        if c + 1 < _GN:
            off = base + (c + 1) * _GCHUNK
            pltpu.sync_copy(tok_hbm.at[pl.ds(off, _GCHUNK)], idx_v.at[nxt])
        g_prev.wait()
        w_handles[cur] = pltpu.async_copy(
            rows_v.at[cur], x_hbm.at[pl.ds(base + c * _GCHUNK, _GCHUNK)], wsem)
        if c + 1 < _GN:
            if w_handles[nxt] is not None:
                w_handles[nxt].wait()
            g_prev = pltpu.async_copy(hid_hbm.at[idx_v.at[nxt]], rows_v.at[nxt],
                                      gsem)
    w_handles[(_GN - 2) % 2].wait()
    w_handles[(_GN - 1) % 2].wait()


def _sc_gather(tok_sorted, hidden_states):
    mesh = plsc.VectorSubcoreMesh(core_axis_name="c", subcore_axis_name="s")
    fn = functools.partial(
        pl.kernel,
        mesh=mesh,
        out_type=jax.ShapeDtypeStruct((_C, _H), jnp.float32),
        scratch_types=[
            pltpu.VMEM((2, _GCHUNK), jnp.int32),
            pltpu.VMEM((2, _GCHUNK, _H), jnp.float32),
            pltpu.SemaphoreType.DMA,
            pltpu.SemaphoreType.DMA,
        ],
    )(_sc_gather_body)
    return fn(tok_sorted, hidden_states)


def _mlp_body(be_ref, bv_ref, x_ref, g_ref, u_ref, d_ref, w_ref, out_ref):
    s = pl.program_id(0)
    j = pl.program_id(1)

    @pl.when(bv_ref[s] == 1)
    def _():
        @pl.when(j == 0)
        def _():
            out_ref[...] = jnp.zeros_like(out_ref)

        x = x_ref[...]
        g = lax.dot_general(x, g_ref[0], (((1,), (1,)), ((), ())),
                            preferred_element_type=jnp.float32)
        u = lax.dot_general(x, u_ref[0], (((1,), (1,)), ((), ())),
                            preferred_element_type=jnp.float32)
        h = g * lax.logistic(g) * u
        h = h * w_ref[...]
        out_ref[...] += lax.dot_general(h, d_ref[0], (((1,), (1,)), ((), ())),
                                        preferred_element_type=jnp.float32)


def _tc_grouped_mlp(block_expert, block_valid, x_sorted, gate_up_proj,
                    down_proj, w_sorted):
    w2 = w_sorted.reshape(_C, 1)

    def gmap(s, j, be, bv):
        return (be[s], jnp.where(bv[s] == 1, j, _JB - 1), 0)

    def umap(s, j, be, bv):
        return (be[s], _JB + jnp.where(bv[s] == 1, j, _JB - 1), 0)

    def dmap(s, j, be, bv):
        return (be[s], 0, jnp.where(bv[s] == 1, j, _JB - 1))

    grid_spec = pltpu.PrefetchScalarGridSpec(
        num_scalar_prefetch=2,
        grid=(_NB, _JB),
        in_specs=[
            pl.BlockSpec((_B, _H), lambda s, j, be, bv: (s, 0)),
            pl.BlockSpec((1, _IB, _H), gmap),
            pl.BlockSpec((1, _IB, _H), umap),
            pl.BlockSpec((1, _H, _IB), dmap),
            pl.BlockSpec((_B, 1), lambda s, j, be, bv: (s, 0)),
        ],
        out_specs=pl.BlockSpec((_B, _H), lambda s, j, be, bv: (s, 0)),
    )
    return pl.pallas_call(
        _mlp_body,
        grid_spec=grid_spec,
        out_shape=jax.ShapeDtypeStruct((_C, _H), jnp.float32),
    )(block_expert, block_valid, x_sorted, gate_up_proj, gate_up_proj,
      down_proj, w2)


_TOK_PER_W = _T // _NW
_CCHUNK = 16


def _sc_combine_body(p0_hbm, p1_hbm, osort_hbm, fin_hbm,
                     i0_v, i1_v, r0_v, r1_v, sem0, sem1):
    wid = lax.axis_index("s") * 2 + lax.axis_index("c")
    base = wid * _TOK_PER_W
    for c in range(_TOK_PER_W // _CCHUNK):
        off = base + c * _CCHUNK
        pltpu.sync_copy(p0_hbm.at[pl.ds(off, _CCHUNK)], i0_v)
        pltpu.sync_copy(p1_hbm.at[pl.ds(off, _CCHUNK)], i1_v)
        cp0 = pltpu.async_copy(osort_hbm.at[i0_v], r0_v, sem0)
        cp1 = pltpu.async_copy(osort_hbm.at[i1_v], r1_v, sem1)
        cp0.wait()
        cp1.wait()

        def row_body(r, carry):
            for cc in range(_H // 16):
                sl = pl.ds(cc * 16, 16)
                r0_v[r, sl] = r0_v[r, sl] + r1_v[r, sl]
            return carry

        lax.fori_loop(0, _CCHUNK, row_body, 0)
        pltpu.sync_copy(r0_v, fin_hbm.at[pl.ds(off, _CCHUNK)])


def _sc_combine(pos0, pos1, out_sorted):
    mesh = plsc.VectorSubcoreMesh(core_axis_name="c", subcore_axis_name="s")
    fn = functools.partial(
        pl.kernel,
        mesh=mesh,
        out_type=jax.ShapeDtypeStruct((_T, _H), jnp.float32),
        scratch_types=[
            pltpu.VMEM((_CCHUNK,), jnp.int32),
            pltpu.VMEM((_CCHUNK,), jnp.int32),
            pltpu.VMEM((_CCHUNK, _H), jnp.float32),
            pltpu.VMEM((_CCHUNK, _H), jnp.float32),
            pltpu.SemaphoreType.DMA,
            pltpu.SemaphoreType.DMA,
        ],
    )(_sc_combine_body)
    return fn(pos0, pos1, out_sorted)


def kernel(hidden_states, top_k_index, top_k_weights, gate_up_proj, down_proj,
           per_expert_scale):
    (tok_sorted, w_sorted, block_expert, block_valid,
     pos0, pos1) = _routing_metadata(top_k_index, top_k_weights,
                                     per_expert_scale)
    x_sorted = _sc_gather(tok_sorted, hidden_states)
    out_sorted = _tc_grouped_mlp(block_expert, block_valid, x_sorted,
                                 gate_up_proj, down_proj, w_sorted)
    return _sc_combine(pos0, pos1, out_sorted)


# speedup vs baseline: 1.4042x; 1.3711x over previous
import functools

import jax
import jax.numpy as jnp
from jax import lax
from jax.experimental import pallas as pl
from jax.experimental.pallas import tpu as pltpu
from jax.experimental.pallas import tpu_sc as plsc

_E = 8
_I = 4096
_H = 2048
_T = 2048
_K = 2
_A = _T * _K

_B = 512
_C = _A + _E * _B
_NB = _C // _B
_IB = 256
_JB = _I // _IB

_NW = 32


def _routing_metadata(top_k_index, top_k_weights, per_expert_scale):
    e_flat = top_k_index.reshape(-1).astype(jnp.int32)
    oh = (e_flat[:, None] == jnp.arange(_E, dtype=jnp.int32)[None, :])
    oh_i = oh.astype(jnp.int32)
    w_flat = (top_k_weights.reshape(-1)
              * (oh.astype(jnp.float32) @ per_expert_scale))
    ranks = jnp.cumsum(oh_i, axis=0) - oh_i
    counts = jnp.sum(oh_i, axis=0)
    blocks_per_e = (counts + _B - 1) // _B
    block_off = jnp.cumsum(blocks_per_e) - blocks_per_e
    total_blocks = block_off[-1] + blocks_per_e[-1]
    pos = (block_off[e_flat] * _B
           + jnp.sum(ranks * oh_i, axis=1)).astype(jnp.int32)
    tok = (jnp.arange(_A, dtype=jnp.int32) // _K)
    tok_sorted = jnp.zeros((_C,), jnp.int32).at[pos].set(tok)
    w_sorted = jnp.zeros((_C,), jnp.float32).at[pos].set(w_flat)
    bgrid = jnp.arange(_NB, dtype=jnp.int32)
    block_expert = (jnp.sum(bgrid[:, None] >= block_off[None, :], axis=1)
                    .astype(jnp.int32) - 1)
    block_valid = (bgrid < total_blocks).astype(jnp.int32)
    pos2 = pos.reshape(_T, _K)
    return tok_sorted, w_sorted, block_expert, block_valid, pos2[:, 0], pos2[:, 1]


def _mlp_body(be_ref, bv_ref, tok_ref, hid_ref, g_ref, u_ref, d_ref, w_ref,
              out_ref, x_ref):
    s = pl.program_id(0)
    j = pl.program_id(1)

    @pl.when(bv_ref[s] == 1)
    def _():
        @pl.when(j == 0)
        def _():
            tok = tok_ref[...]
            toks = lax.broadcasted_iota(jnp.int32, (_B, _T), 1)
            p = (toks == tok).astype(jnp.float32)
            x_ref[...] = lax.dot_general(
                p, hid_ref[...], (((1,), (0,)), ((), ())),
                preferred_element_type=jnp.float32)
            out_ref[...] = jnp.zeros_like(out_ref)

        x = x_ref[...]
        g = lax.dot_general(x, g_ref[0], (((1,), (1,)), ((), ())),
                            preferred_element_type=jnp.float32)
        u = lax.dot_general(x, u_ref[0], (((1,), (1,)), ((), ())),
                            preferred_element_type=jnp.float32)
        h = g * lax.logistic(g) * u
        h = h * w_ref[...]
        out_ref[...] += lax.dot_general(h, d_ref[0], (((1,), (1,)), ((), ())),
                                        preferred_element_type=jnp.float32)


def _tc_grouped_mlp(block_expert, block_valid, tok_sorted, hidden_states,
                    gate_up_proj, down_proj, w_sorted):
    w2 = w_sorted.reshape(_C, 1)
    tok2 = tok_sorted.reshape(_C, 1)

    def gmap(s, j, be, bv):
        return (be[s], jnp.where(bv[s] == 1, j, _JB - 1), 0)

    def umap(s, j, be, bv):
        return (be[s], _JB + jnp.where(bv[s] == 1, j, _JB - 1), 0)

    def dmap(s, j, be, bv):
        return (be[s], 0, jnp.where(bv[s] == 1, j, _JB - 1))

    grid_spec = pltpu.PrefetchScalarGridSpec(
        num_scalar_prefetch=2,
        grid=(_NB, _JB),
        in_specs=[
            pl.BlockSpec((_B, 1), lambda s, j, be, bv: (s, 0)),
            pl.BlockSpec((_T, _H), lambda s, j, be, bv: (0, 0)),
            pl.BlockSpec((1, _IB, _H), gmap),
            pl.BlockSpec((1, _IB, _H), umap),
            pl.BlockSpec((1, _H, _IB), dmap),
            pl.BlockSpec((_B, 1), lambda s, j, be, bv: (s, 0)),
        ],
        out_specs=pl.BlockSpec((_B, _H), lambda s, j, be, bv: (s, 0)),
        scratch_shapes=[pltpu.VMEM((_B, _H), jnp.float32)],
    )
    return pl.pallas_call(
        _mlp_body,
        grid_spec=grid_spec,
        out_shape=jax.ShapeDtypeStruct((_C, _H), jnp.float32),
    )(block_expert, block_valid, tok2, hidden_states, gate_up_proj,
      gate_up_proj, down_proj, w2)


_TOK_PER_W = _T // _NW
_CCHUNK = 16


def _sc_combine_body(p0_hbm, p1_hbm, osort_hbm, fin_hbm,
                     i0_v, i1_v, r0_v, r1_v, sem0, sem1):
    wid = lax.axis_index("s") * 2 + lax.axis_index("c")
    base = wid * _TOK_PER_W
    for c in range(_TOK_PER_W // _CCHUNK):
        off = base + c * _CCHUNK
        pltpu.sync_copy(p0_hbm.at[pl.ds(off, _CCHUNK)], i0_v)
        pltpu.sync_copy(p1_hbm.at[pl.ds(off, _CCHUNK)], i1_v)
        cp0 = pltpu.async_copy(osort_hbm.at[i0_v], r0_v, sem0)
        cp1 = pltpu.async_copy(osort_hbm.at[i1_v], r1_v, sem1)
        cp0.wait()
        cp1.wait()

        def row_body(r, carry):
            for cc in range(_H // 16):
                sl = pl.ds(cc * 16, 16)
                r0_v[r, sl] = r0_v[r, sl] + r1_v[r, sl]
            return carry

        lax.fori_loop(0, _CCHUNK, row_body, 0)
        pltpu.sync_copy(r0_v, fin_hbm.at[pl.ds(off, _CCHUNK)])


def _sc_combine(pos0, pos1, out_sorted):
    mesh = plsc.VectorSubcoreMesh(core_axis_name="c", subcore_axis_name="s")
    fn = functools.partial(
        pl.kernel,
        mesh=mesh,
        out_type=jax.ShapeDtypeStruct((_T, _H), jnp.float32),
        scratch_types=[
            pltpu.VMEM((_CCHUNK,), jnp.int32),
            pltpu.VMEM((_CCHUNK,), jnp.int32),
            pltpu.VMEM((_CCHUNK, _H), jnp.float32),
            pltpu.VMEM((_CCHUNK, _H), jnp.float32),
            pltpu.SemaphoreType.DMA,
            pltpu.SemaphoreType.DMA,
        ],
    )(_sc_combine_body)
    return fn(pos0, pos1, out_sorted)


def kernel(hidden_states, top_k_index, top_k_weights, gate_up_proj, down_proj,
           per_expert_scale):
    (tok_sorted, w_sorted, block_expert, block_valid,
     pos0, pos1) = _routing_metadata(top_k_index, top_k_weights,
                                     per_expert_scale)
    out_sorted = _tc_grouped_mlp(block_expert, block_valid, tok_sorted,
                                 hidden_states, gate_up_proj, down_proj,
                                 w_sorted)
    return _sc_combine(pos0, pos1, out_sorted)


# speedup vs baseline: 1.5274x; 1.0878x over previous
import functools

import jax
import jax.numpy as jnp
from jax import lax
from jax.experimental import pallas as pl
from jax.experimental.pallas import tpu as pltpu
from jax.experimental.pallas import tpu_sc as plsc

_E = 8
_I = 4096
_H = 2048
_T = 2048
_K = 2
_A = _T * _K

_B = 512
_C = _A + _E * _B
_NB = _C // _B
_IB = 512
_JB = _I // _IB

_NW = 32


def _routing_metadata(top_k_index, top_k_weights, per_expert_scale):
    e_flat = top_k_index.reshape(-1).astype(jnp.int32)
    oh = (e_flat[:, None] == jnp.arange(_E, dtype=jnp.int32)[None, :])
    oh_i = oh.astype(jnp.int32)
    w_flat = (top_k_weights.reshape(-1)
              * (oh.astype(jnp.float32) @ per_expert_scale))
    ranks = jnp.cumsum(oh_i, axis=0) - oh_i
    counts = jnp.sum(oh_i, axis=0)
    blocks_per_e = (counts + _B - 1) // _B
    block_off = jnp.cumsum(blocks_per_e) - blocks_per_e
    total_blocks = block_off[-1] + blocks_per_e[-1]
    pos = (block_off[e_flat] * _B
           + jnp.sum(ranks * oh_i, axis=1)).astype(jnp.int32)
    tok = (jnp.arange(_A, dtype=jnp.int32) // _K)
    tok_sorted = jnp.zeros((_C,), jnp.int32).at[pos].set(tok)
    w_sorted = jnp.zeros((_C,), jnp.float32).at[pos].set(w_flat)
    bgrid = jnp.arange(_NB, dtype=jnp.int32)
    block_expert = (jnp.sum(bgrid[:, None] >= block_off[None, :], axis=1)
                    .astype(jnp.int32) - 1)
    block_valid = (bgrid < total_blocks).astype(jnp.int32)
    pos2 = pos.reshape(_T, _K)
    return tok_sorted, w_sorted, block_expert, block_valid, pos2[:, 0], pos2[:, 1]


def _mlp_body(be_ref, bv_ref, tok_ref, hid_ref, g_ref, u_ref, d_ref, w_ref,
              out_ref, x_ref):
    s = pl.program_id(0)
    j = pl.program_id(1)

    @pl.when(bv_ref[s] == 1)
    def _():
        @pl.when(j == 0)
        def _():
            tok = tok_ref[...]
            toks = lax.broadcasted_iota(jnp.int32, (_B, _T), 1)
            p = (toks == tok).astype(jnp.float32)
            x_ref[...] = lax.dot_general(
                p, hid_ref[...], (((1,), (0,)), ((), ())),
                preferred_element_type=jnp.float32)
            out_ref[...] = jnp.zeros_like(out_ref)

        x = x_ref[...]
        g = lax.dot_general(x, g_ref[0], (((1,), (1,)), ((), ())),
                            preferred_element_type=jnp.float32)
        u = lax.dot_general(x, u_ref[0], (((1,), (1,)), ((), ())),
                            preferred_element_type=jnp.float32)
        h = g * lax.logistic(g) * u
        h = h * w_ref[...]
        out_ref[...] += lax.dot_general(h, d_ref[0], (((1,), (1,)), ((), ())),
                                        preferred_element_type=jnp.float32)


def _tc_grouped_mlp(block_expert, block_valid, tok_sorted, hidden_states,
                    gate_up_proj, down_proj, w_sorted):
    w2 = w_sorted.reshape(_C, 1)
    tok2 = tok_sorted.reshape(_C, 1)

    def gmap(s, j, be, bv):
        return (be[s], jnp.where(bv[s] == 1, j, _JB - 1), 0)

    def umap(s, j, be, bv):
        return (be[s], _JB + jnp.where(bv[s] == 1, j, _JB - 1), 0)

    def dmap(s, j, be, bv):
        return (be[s], 0, jnp.where(bv[s] == 1, j, _JB - 1))

    grid_spec = pltpu.PrefetchScalarGridSpec(
        num_scalar_prefetch=2,
        grid=(_NB, _JB),
        in_specs=[
            pl.BlockSpec((_B, 1), lambda s, j, be, bv: (s, 0)),
            pl.BlockSpec((_T, _H), lambda s, j, be, bv: (0, 0)),
            pl.BlockSpec((1, _IB, _H), gmap),
            pl.BlockSpec((1, _IB, _H), umap),
            pl.BlockSpec((1, _H, _IB), dmap),
            pl.BlockSpec((_B, 1), lambda s, j, be, bv: (s, 0)),
        ],
        out_specs=pl.BlockSpec((_B, _H), lambda s, j, be, bv: (s, 0)),
        scratch_shapes=[pltpu.VMEM((_B, _H), jnp.float32)],
    )
    return pl.pallas_call(
        _mlp_body,
        grid_spec=grid_spec,
        out_shape=jax.ShapeDtypeStruct((_C, _H), jnp.float32),
    )(block_expert, block_valid, tok2, hidden_states, gate_up_proj,
      gate_up_proj, down_proj, w2)


_TOK_PER_W = _T // _NW
_CCHUNK = 16


def _sc_combine_body(p0_hbm, p1_hbm, osort_hbm, fin_hbm,
                     i0_v, i1_v, r0_v, r1_v, sem0, sem1):
    wid = lax.axis_index("s") * 2 + lax.axis_index("c")
    base = wid * _TOK_PER_W
    for c in range(_TOK_PER_W // _CCHUNK):
        off = base + c * _CCHUNK
        pltpu.sync_copy(p0_hbm.at[pl.ds(off, _CCHUNK)], i0_v)
        pltpu.sync_copy(p1_hbm.at[pl.ds(off, _CCHUNK)], i1_v)
        cp0 = pltpu.async_copy(osort_hbm.at[i0_v], r0_v, sem0)
        cp1 = pltpu.async_copy(osort_hbm.at[i1_v], r1_v, sem1)
        cp0.wait()
        cp1.wait()

        def row_body(r, carry):
            for cc in range(_H // 16):
                sl = pl.ds(cc * 16, 16)
                r0_v[r, sl] = r0_v[r, sl] + r1_v[r, sl]
            return carry

        lax.fori_loop(0, _CCHUNK, row_body, 0)
        pltpu.sync_copy(r0_v, fin_hbm.at[pl.ds(off, _CCHUNK)])


def _sc_combine(pos0, pos1, out_sorted):
    mesh = plsc.VectorSubcoreMesh(core_axis_name="c", subcore_axis_name="s")
    fn = functools.partial(
        pl.kernel,
        mesh=mesh,
        out_type=jax.ShapeDtypeStruct((_T, _H), jnp.float32),
        scratch_types=[
            pltpu.VMEM((_CCHUNK,), jnp.int32),
            pltpu.VMEM((_CCHUNK,), jnp.int32),
            pltpu.VMEM((_CCHUNK, _H), jnp.float32),
            pltpu.VMEM((_CCHUNK, _H), jnp.float32),
            pltpu.SemaphoreType.DMA,
            pltpu.SemaphoreType.DMA,
        ],
    )(_sc_combine_body)
    return fn(pos0, pos1, out_sorted)


def kernel(hidden_states, top_k_index, top_k_weights, gate_up_proj, down_proj,
           per_expert_scale):
    (tok_sorted, w_sorted, block_expert, block_valid,
     pos0, pos1) = _routing_metadata(top_k_index, top_k_weights,
                                     per_expert_scale)
    out_sorted = _tc_grouped_mlp(block_expert, block_valid, tok_sorted,
                                 hidden_states, gate_up_proj, down_proj,
                                 w_sorted)
    return _sc_combine(pos0, pos1, out_sorted)
